# Initial kernel scaffold; baseline (speedup 1.0000x reference)
import jax
import jax.numpy as jnp
from jax.experimental import pallas as pl


def kernel(node_feats_0, node_feats_1, edge_feats_0, edge_index, basis, W1, b1, ln1_g, ln1_b, W2, b2, ln2_g, ln2_b, W3, k_self_0, k_self_1):
    raise NotImplementedError("write your pallas kernel here")


# speedup vs baseline: 7.1743x; 7.1743x over previous
import functools

import jax
import jax.numpy as jnp
import numpy as np
from jax import lax
from jax.experimental import pallas as pl
from jax.experimental.pallas import tpu as pltpu
from jax.experimental.pallas import tpu_sc as plsc

_N = 8192
_E = 65536
_C = 16
_MID = 32
_F = 6
_K = 4
_BE = 1024
_CH = 128


def _ln(x, g, b):
    mu = jnp.mean(x, axis=-1, keepdims=True)
    var = jnp.mean((x - mu) ** 2, axis=-1, keepdims=True)
    return (x - mu) / jnp.sqrt(var + 1e-5) * g + b


def _self_table_body(nfT_ref, KS_ref, s_ref):
    s_ref[...] = jnp.dot(nfT_ref[...], KS_ref[...],
                         preferred_element_type=jnp.float32)


def _self_table(nfT, KS):
    nb = _N // _BE
    return pl.pallas_call(
        _self_table_body,
        grid=(nb,),
        in_specs=[
            pl.BlockSpec((_BE, 64), lambda g: (g, 0)),
            pl.BlockSpec((64, 64), lambda g: (0, 0)),
        ],
        out_specs=pl.BlockSpec((_BE, 64), lambda g: (g, 0)),
        out_shape=jax.ShapeDtypeStruct((_N, 64), jnp.float32),
    )(nfT, KS)


def _sc_gather(nfT, s, src, dst):
    mesh = plsc.VectorSubcoreMesh(core_axis_name="c", subcore_axis_name="s")
    nc, ns = mesh.num_cores, mesh.num_subcores
    nw = nc * ns
    per_w = _E // nw
    n_ch = per_w // _CH

    @functools.partial(
        pl.kernel,
        mesh=mesh,
        compiler_params=pltpu.CompilerParams(use_tc_tiling_on_sc=False),
        out_type=(
            jax.ShapeDtypeStruct((_E, 64), jnp.float32),
            jax.ShapeDtypeStruct((_E, 64), jnp.float32),
        ),
        scratch_types=[
            pltpu.VMEM((_CH,), jnp.int32),
            pltpu.VMEM((_CH,), jnp.int32),
            pltpu.VMEM((_CH, 64), jnp.float32),
            pltpu.VMEM((_CH, 64), jnp.float32),
            pltpu.SemaphoreType.DMA,
        ],
    )
    def k(nfT_h, s_h, src_h, dst_h, feat_h, se_h, si_v, di_v, fr_v, sr_v, sem):
        wid = lax.axis_index("s") * nc + lax.axis_index("c")
        base = wid * per_w

        def body(i, carry):
            off = base + i * _CH
            pltpu.sync_copy(src_h.at[pl.ds(off, _CH)], si_v)
            pltpu.sync_copy(dst_h.at[pl.ds(off, _CH)], di_v)
            pltpu.async_copy(nfT_h.at[si_v], fr_v, sem).wait()
            pltpu.async_copy(s_h.at[di_v], sr_v, sem).wait()
            pltpu.sync_copy(fr_v, feat_h.at[pl.ds(off, _CH)])
            pltpu.sync_copy(sr_v, se_h.at[pl.ds(off, _CH)])
            return carry

        lax.fori_loop(0, n_ch, body, 0)

    return k(nfT, s, src, dst)


def _edge_body(inv_ref, feat_ref, se_ref, bF_ref, W1_ref, b1_ref, g1_ref,
               be1_ref, W2_ref, b2_ref, g2_ref, be2_ref, W3_ref, REP_ref,
               S96_ref, PK_ref, out_ref):
    h = jnp.dot(inv_ref[...], W1_ref[...],
                preferred_element_type=jnp.float32) + b1_ref[...]
    h = jnp.maximum(_ln(h, g1_ref[...], be1_ref[...]), 0.0)
    h = jnp.dot(h, W2_ref[...],
                preferred_element_type=jnp.float32) + b2_ref[...]
    h = jnp.maximum(_ln(h, g2_ref[...], be2_ref[...]), 0.0)
    RW = jnp.dot(h, W3_ref[...], preferred_element_type=jnp.float32)

    feat = feat_ref[...]
    bF = bF_ref[...]
    REP = REP_ref[...]
    PK = PK_ref[...]

    Fl = [jnp.dot(feat[:, l * 16:(l + 1) * 16], REP,
                  preferred_element_type=jnp.float32) for l in range(_K)]

    acc = se_ref[...]
    for k in range(_K):
        T = None
        for l in range(_K):
            b6 = bF[:, l * 24 + k * 6: l * 24 + k * 6 + 6]
            b96 = jnp.concatenate([b6] * _C, axis=1)
            t = Fl[l] * b96
            T = t if T is None else T + t
        T1536 = jnp.concatenate([T] * _C, axis=1)
        Gk = RW * T1536
        ok = jnp.dot(Gk, S96_ref[...],
                     preferred_element_type=jnp.float32)
        acc = acc + jnp.dot(ok, PK[k * 16:(k + 1) * 16, :],
                            preferred_element_type=jnp.float32)
    out_ref[...] = acc


def _edge_dense(inv, feat_e, se_e, bF, W1, b1, g1, be1, W2, b2, g2, be2, W3,
                REP, S96, PK):
    nb = _E // _BE
    edge = lambda w: pl.BlockSpec((_BE, w), lambda g: (g, 0))
    full = lambda a, b: pl.BlockSpec((a, b), lambda g: (0, 0))
    return pl.pallas_call(
        _edge_body,
        grid=(nb,),
        in_specs=[
            edge(5), edge(64), edge(64), edge(96),
            full(5, 32), full(1, 32), full(1, 32), full(1, 32),
            full(32, 32), full(1, 32), full(1, 32), full(1, 32),
            full(32, 1536), full(16, 96), full(1536, 16), full(64, 64),
        ],
        out_specs=edge(64),
        out_shape=jax.ShapeDtypeStruct((_E, 64), jnp.float32),
    )(inv, feat_e, se_e, bF, W1, b1, g1, be1, W2, b2, g2, be2, W3, REP, S96,
      PK)


def _sc_scatter(out_e, dst, zeros_hbm):
    mesh = plsc.VectorSubcoreMesh(core_axis_name="c", subcore_axis_name="s")
    nc, ns = mesh.num_cores, mesh.num_subcores
    nw = nc * ns
    per_w = _E // nw
    n_ch = per_w // _CH
    rows_per_tile = _N // ns

    @functools.partial(
        pl.kernel,
        mesh=mesh,
        compiler_params=pltpu.CompilerParams(use_tc_tiling_on_sc=False),
        out_type=jax.ShapeDtypeStruct((nc, _N, 64), jnp.float32),
        scratch_types=[
            pltpu.VMEM((_CH,), jnp.int32),
            pltpu.VMEM((_CH, 64), jnp.float32),
            pltpu.VMEM_SHARED((_N, 64), jnp.float32),
        ],
    )
    def k(oe_h, dst_h, z_h, out_h, di_v, rows_v, acc_sh):
        cid = lax.axis_index("c")
        sid = lax.axis_index("s")
        wid = sid * nc + cid
        rbase = sid * rows_per_tile
        pltpu.sync_copy(z_h.at[pl.ds(rbase, rows_per_tile)],
                        acc_sh.at[pl.ds(rbase, rows_per_tile)])
        plsc.subcore_barrier()

        def body(i, carry):
            off = wid * per_w + i * _CH
            pltpu.sync_copy(dst_h.at[pl.ds(off, _CH)], di_v)
            pltpu.sync_copy(oe_h.at[pl.ds(off, _CH)], rows_v)
            pltpu.sync_copy(rows_v, acc_sh.at[di_v], add=True)
            return carry

        lax.fori_loop(0, n_ch, body, 0)
        plsc.subcore_barrier()
        pltpu.sync_copy(acc_sh.at[pl.ds(rbase, rows_per_tile)],
                        out_h.at[cid].at[pl.ds(rbase, rows_per_tile)])

    return k(out_e, dst, zeros_hbm)


def _sum_partials_body(p_ref, out_ref):
    out_ref[...] = p_ref[0] + p_ref[1]


def _sum_partials(p):
    nb = _N // _BE
    return pl.pallas_call(
        _sum_partials_body,
        grid=(nb,),
        in_specs=[pl.BlockSpec((2, _BE, 64), lambda g: (0, g, 0))],
        out_specs=pl.BlockSpec((_BE, 64), lambda g: (g, 0)),
        out_shape=jax.ShapeDtypeStruct((_N, 64), jnp.float32),
    )(p)


def _constants():
    REP = np.zeros((_C, _C * _F), np.float32)
    for c in range(_C):
        for f in range(_F):
            REP[c, c * _F + f] = 1.0
    S96 = np.zeros((_C * _C * _F, _C), np.float32)
    for o in range(_C):
        S96[o * 96:(o + 1) * 96, o] = 1.0
    PK = np.zeros((64, 64), np.float32)
    for k in range(_K):
        for o in range(_C):
            PK[k * 16 + o, o * 4 + k] = 1.0
    return jnp.asarray(REP), jnp.asarray(S96), jnp.asarray(PK)


def kernel(node_feats_0, node_feats_1, edge_feats_0, edge_index, basis, W1,
           b1, ln1_g, ln1_b, W2, b2, ln2_g, ln2_b, W3, k_self_0, k_self_1):
    src = edge_index[0]
    dst = edge_index[1]
    inv = edge_feats_0[:, :, 0]

    nf_cat = jnp.concatenate([node_feats_0, node_feats_1], axis=-1)
    nfT = jnp.transpose(nf_cat, (0, 2, 1)).reshape(_N, 64)

    bF = jnp.transpose(basis, (0, 1, 3, 2)).reshape(_E, 96)

    KS = jnp.zeros((64, 64), jnp.float32)
    for k in range(_K):
        ks = k_self_0 if k == 0 else k_self_1
        KS = KS.at[k * 16:(k + 1) * 16, k::4].set(ks.T)

    REP, S96, PK = _constants()

    s = _self_table(nfT, KS)
    feat_e, se_e = _sc_gather(nfT, s, src, dst)
    out_e = _edge_dense(inv, feat_e, se_e, bF, W1, b1.reshape(1, 32),
                        ln1_g.reshape(1, 32), ln1_b.reshape(1, 32), W2,
                        b2.reshape(1, 32), ln2_g.reshape(1, 32),
                        ln2_b.reshape(1, 32), W3, REP, S96, PK)
    p = _sc_scatter(out_e, dst, jnp.zeros((_N, 64), jnp.float32))
    res = _sum_partials(p)
    return res.reshape(_N, _C, _K)


# speedup vs baseline: 10.1558x; 1.4156x over previous
import functools

import jax
import jax.numpy as jnp
import numpy as np
from jax import lax
from jax.experimental import pallas as pl
from jax.experimental.pallas import tpu as pltpu
from jax.experimental.pallas import tpu_sc as plsc

_N = 8192
_E = 65536
_C = 16
_MID = 32
_F = 6
_K = 4
_BE = 1024
_CH = 128


def _ln(x, g, b):
    mu = jnp.mean(x, axis=-1, keepdims=True)
    var = jnp.mean((x - mu) ** 2, axis=-1, keepdims=True)
    return (x - mu) / jnp.sqrt(var + 1e-5) * g + b


def _self_table_body(nfT_ref, KS_ref, s_ref):
    s_ref[...] = jnp.dot(nfT_ref[...], KS_ref[...],
                         preferred_element_type=jnp.float32)


def _self_table(nfT, KS):
    nb = _N // _BE
    return pl.pallas_call(
        _self_table_body,
        grid=(nb,),
        in_specs=[
            pl.BlockSpec((_BE, 64), lambda g: (g, 0)),
            pl.BlockSpec((64, 64), lambda g: (0, 0)),
        ],
        out_specs=pl.BlockSpec((_BE, 64), lambda g: (g, 0)),
        out_shape=jax.ShapeDtypeStruct((_N, 64), jnp.float32),
    )(nfT, KS)


def _sc_gather(nfT, s, src, dst):
    mesh = plsc.VectorSubcoreMesh(core_axis_name="c", subcore_axis_name="s")
    nc, ns = mesh.num_cores, mesh.num_subcores
    nw = nc * ns
    per_w = _E // nw
    n_ch = per_w // _CH

    @functools.partial(
        pl.kernel,
        mesh=mesh,
        compiler_params=pltpu.CompilerParams(use_tc_tiling_on_sc=False),
        out_type=(
            jax.ShapeDtypeStruct((_E, 64), jnp.float32),
            jax.ShapeDtypeStruct((_E, 64), jnp.float32),
        ),
        scratch_types=[
            pltpu.VMEM((_CH,), jnp.int32),
            pltpu.VMEM((_CH,), jnp.int32),
            pltpu.VMEM((_CH, 64), jnp.float32),
            pltpu.VMEM((_CH, 64), jnp.float32),
            pltpu.SemaphoreType.DMA,
        ],
    )
    def k(nfT_h, s_h, src_h, dst_h, feat_h, se_h, si_v, di_v, fr_v, sr_v, sem):
        wid = lax.axis_index("s") * nc + lax.axis_index("c")
        base = wid * per_w

        def body(i, carry):
            off = base + i * _CH
            pltpu.sync_copy(src_h.at[pl.ds(off, _CH)], si_v)
            pltpu.sync_copy(dst_h.at[pl.ds(off, _CH)], di_v)
            pltpu.async_copy(nfT_h.at[si_v], fr_v, sem).wait()
            pltpu.async_copy(s_h.at[di_v], sr_v, sem).wait()
            pltpu.sync_copy(fr_v, feat_h.at[pl.ds(off, _CH)])
            pltpu.sync_copy(sr_v, se_h.at[pl.ds(off, _CH)])
            return carry

        lax.fori_loop(0, n_ch, body, 0)

    return k(nfT, s, src, dst)


def _edge_body(inv_ref, feat_ref, se_ref, bF_ref, W1_ref, b1_ref, g1_ref,
               be1_ref, W2_ref, b2_ref, g2_ref, be2_ref, W3T_ref, REP_ref,
               REPM_ref, RED_ref, PK_ref, TILEB_ref, out_ref):
    h = jnp.dot(inv_ref[...], W1_ref[...],
                preferred_element_type=jnp.float32) + b1_ref[...]
    h = jnp.maximum(_ln(h, g1_ref[...], be1_ref[...]), 0.0)
    h = jnp.dot(h, W2_ref[...],
                preferred_element_type=jnp.float32) + b2_ref[...]
    h = jnp.maximum(_ln(h, g2_ref[...], be2_ref[...]), 0.0)
    hrep = jnp.dot(h, REPM_ref[...], preferred_element_type=jnp.float32)

    feat = feat_ref[...]
    bF = bF_ref[...]
    REP = REP_ref[...]
    PK = PK_ref[...]

    Fl = [jnp.dot(feat[:, l * 16:(l + 1) * 16], REP,
                  preferred_element_type=jnp.float32) for l in range(_K)]
    BT = jnp.dot(bF, TILEB_ref[...], preferred_element_type=jnp.float32)

    acc = se_ref[...]
    for k in range(_K):
        T = None
        for l in range(_K):
            j = (l * 4 + k) * 96
            t = Fl[l] * BT[:, j:j + 96]
            T = t if T is None else T + t
        S = jnp.dot(T, W3T_ref[...],
                    preferred_element_type=jnp.float32)
        ok = jnp.dot(S * hrep, RED_ref[...],
                     preferred_element_type=jnp.float32)
        acc = acc + jnp.dot(ok, PK[k * 16:(k + 1) * 16, :],
                            preferred_element_type=jnp.float32)
    out_ref[...] = acc


def _edge_dense(inv, feat_e, se_e, bF, W1, b1, g1, be1, W2, b2, g2, be2, W3T,
                REP, REPM, RED, PK, TILEB):
    nb = _E // _BE
    edge = lambda w: pl.BlockSpec((_BE, w), lambda g: (g, 0))
    full = lambda a, b: pl.BlockSpec((a, b), lambda g: (0, 0))
    return pl.pallas_call(
        _edge_body,
        grid=(nb,),
        in_specs=[
            edge(5), edge(64), edge(64), edge(96),
            full(5, 32), full(1, 32), full(1, 32), full(1, 32),
            full(32, 32), full(1, 32), full(1, 32), full(1, 32),
            full(96, 512), full(16, 96), full(32, 512), full(512, 16),
            full(64, 64), full(96, 1536),
        ],
        out_specs=edge(64),
        out_shape=jax.ShapeDtypeStruct((_E, 64), jnp.float32),
    )(inv, feat_e, se_e, bF, W1, b1, g1, be1, W2, b2, g2, be2, W3T, REP,
      REPM, RED, PK, TILEB)


def _sc_scatter(out_e, dst, zeros_hbm):
    mesh = plsc.VectorSubcoreMesh(core_axis_name="c", subcore_axis_name="s")
    nc, ns = mesh.num_cores, mesh.num_subcores
    nw = nc * ns
    per_w = _E // nw
    n_ch = per_w // _CH
    rows_per_tile = _N // ns

    @functools.partial(
        pl.kernel,
        mesh=mesh,
        compiler_params=pltpu.CompilerParams(use_tc_tiling_on_sc=False),
        out_type=jax.ShapeDtypeStruct((nc, _N, 64), jnp.float32),
        scratch_types=[
            pltpu.VMEM((_CH,), jnp.int32),
            pltpu.VMEM((_CH, 64), jnp.float32),
            pltpu.VMEM_SHARED((_N, 64), jnp.float32),
        ],
    )
    def k(oe_h, dst_h, z_h, out_h, di_v, rows_v, acc_sh):
        cid = lax.axis_index("c")
        sid = lax.axis_index("s")
        wid = sid * nc + cid
        rbase = sid * rows_per_tile
        pltpu.sync_copy(z_h.at[pl.ds(rbase, rows_per_tile)],
                        acc_sh.at[pl.ds(rbase, rows_per_tile)])
        plsc.subcore_barrier()

        def body(i, carry):
            off = wid * per_w + i * _CH
            pltpu.sync_copy(dst_h.at[pl.ds(off, _CH)], di_v)
            pltpu.sync_copy(oe_h.at[pl.ds(off, _CH)], rows_v)
            pltpu.sync_copy(rows_v, acc_sh.at[di_v], add=True)
            return carry

        lax.fori_loop(0, n_ch, body, 0)
        plsc.subcore_barrier()
        pltpu.sync_copy(acc_sh.at[pl.ds(rbase, rows_per_tile)],
                        out_h.at[cid].at[pl.ds(rbase, rows_per_tile)])

    return k(out_e, dst, zeros_hbm)


def _sum_partials_body(p_ref, out_ref):
    out_ref[...] = p_ref[0] + p_ref[1]


def _sum_partials(p):
    nb = _N // _BE
    return pl.pallas_call(
        _sum_partials_body,
        grid=(nb,),
        in_specs=[pl.BlockSpec((2, _BE, 64), lambda g: (0, g, 0))],
        out_specs=pl.BlockSpec((_BE, 64), lambda g: (g, 0)),
        out_shape=jax.ShapeDtypeStruct((_N, 64), jnp.float32),
    )(p)


def _constants():
    REP = np.zeros((_C, _C * _F), np.float32)
    for c in range(_C):
        for f in range(_F):
            REP[c, c * _F + f] = 1.0
    REPM = np.zeros((_MID, _MID * _C), np.float32)
    for m in range(_MID):
        REPM[m, m * 16:(m + 1) * 16] = 1.0
    RED = np.zeros((_MID * _C, _C), np.float32)
    for m in range(_MID):
        for o in range(_C):
            RED[m * 16 + o, o] = 1.0
    PK = np.zeros((64, 64), np.float32)
    for k in range(_K):
        for o in range(_C):
            PK[k * 16 + o, o * 4 + k] = 1.0
    TILEB = np.zeros((96, 16 * 96), np.float32)
    for l in range(_K):
        for k in range(_K):
            for f in range(_F):
                for c in range(_C):
                    TILEB[l * 24 + k * 6 + f, (l * 4 + k) * 96 + c * 6 + f] = 1.0
    return (jnp.asarray(REP), jnp.asarray(REPM), jnp.asarray(RED),
            jnp.asarray(PK), jnp.asarray(TILEB))


def kernel(node_feats_0, node_feats_1, edge_feats_0, edge_index, basis, W1,
           b1, ln1_g, ln1_b, W2, b2, ln2_g, ln2_b, W3, k_self_0, k_self_1):
    src = edge_index[0]
    dst = edge_index[1]
    inv = edge_feats_0[:, :, 0]

    nf_cat = jnp.concatenate([node_feats_0, node_feats_1], axis=-1)
    nfT = jnp.transpose(nf_cat, (0, 2, 1)).reshape(_N, 64)

    bF = jnp.transpose(basis, (0, 1, 3, 2)).reshape(_E, 96)

    KS = jnp.zeros((64, 64), jnp.float32)
    for k in range(_K):
        ks = k_self_0 if k == 0 else k_self_1
        KS = KS.at[k * 16:(k + 1) * 16, k::4].set(ks.T)

    REP, REPM, RED, PK, TILEB = _constants()
    W3T = jnp.transpose(W3.reshape(_MID, _C, 96), (2, 0, 1)).reshape(96, 512)

    s = _self_table(nfT, KS)
    feat_e, se_e = _sc_gather(nfT, s, src, dst)
    out_e = _edge_dense(inv, feat_e, se_e, bF, W1, b1.reshape(1, 32),
                        ln1_g.reshape(1, 32), ln1_b.reshape(1, 32), W2,
                        b2.reshape(1, 32), ln2_g.reshape(1, 32),
                        ln2_b.reshape(1, 32), W3T, REP, REPM, RED, PK, TILEB)
    p = _sc_scatter(out_e, dst, jnp.zeros((_N, 64), jnp.float32))
    res = _sum_partials(p)
    return res.reshape(_N, _C, _K)


# speedup vs baseline: 13.5625x; 1.3355x over previous
import functools

import jax
import jax.numpy as jnp
import numpy as np
from jax import lax
from jax.experimental import pallas as pl
from jax.experimental.pallas import tpu as pltpu
from jax.experimental.pallas import tpu_sc as plsc

_N = 8192
_E = 65536
_C = 16
_MID = 32
_F = 6
_K = 4
_BE = 1024
_CH = 128


def _ln(x, g, b):
    mu = jnp.mean(x, axis=-1, keepdims=True)
    var = jnp.mean((x - mu) ** 2, axis=-1, keepdims=True)
    return (x - mu) / jnp.sqrt(var + 1e-5) * g + b


def _self_table_body(nfT_ref, KS_ref, s_ref):
    s_ref[...] = jnp.dot(nfT_ref[...], KS_ref[...],
                         preferred_element_type=jnp.float32)


def _self_table(nfT, KS):
    nb = _N // _BE
    return pl.pallas_call(
        _self_table_body,
        grid=(nb,),
        in_specs=[
            pl.BlockSpec((_BE, 64), lambda g: (g, 0)),
            pl.BlockSpec((64, 64), lambda g: (0, 0)),
        ],
        out_specs=pl.BlockSpec((_BE, 64), lambda g: (g, 0)),
        out_shape=jax.ShapeDtypeStruct((_N, 64), jnp.float32),
    )(nfT, KS)


def _sc_gather(nfT, s, src, dst):
    mesh = plsc.VectorSubcoreMesh(core_axis_name="c", subcore_axis_name="s")
    nc, ns = mesh.num_cores, mesh.num_subcores
    nw = nc * ns
    per_w = _E // nw
    n_ch = per_w // _CH

    @functools.partial(
        pl.kernel,
        mesh=mesh,
        compiler_params=pltpu.CompilerParams(use_tc_tiling_on_sc=False),
        out_type=(
            jax.ShapeDtypeStruct((_E, 64), jnp.float32),
            jax.ShapeDtypeStruct((_E, 64), jnp.float32),
        ),
        scratch_types=[
            pltpu.VMEM((_CH,), jnp.int32),
            pltpu.VMEM((_CH,), jnp.int32),
            pltpu.VMEM((_CH, 64), jnp.float32),
            pltpu.VMEM((_CH, 64), jnp.float32),
            pltpu.SemaphoreType.DMA,
        ],
    )
    def k(nfT_h, s_h, src_h, dst_h, feat_h, se_h, si_v, di_v, fr_v, sr_v, sem):
        wid = lax.axis_index("s") * nc + lax.axis_index("c")
        base = wid * per_w

        def body(i, carry):
            off = base + i * _CH
            pltpu.sync_copy(src_h.at[pl.ds(off, _CH)], si_v)
            pltpu.sync_copy(dst_h.at[pl.ds(off, _CH)], di_v)
            pltpu.async_copy(nfT_h.at[si_v], fr_v, sem).wait()
            pltpu.async_copy(s_h.at[di_v], sr_v, sem).wait()
            pltpu.sync_copy(fr_v, feat_h.at[pl.ds(off, _CH)])
            pltpu.sync_copy(sr_v, se_h.at[pl.ds(off, _CH)])
            return carry

        lax.fori_loop(0, n_ch, body, 0)

    return k(nfT, s, src, dst)


def _edge_body(inv_ref, feat_ref, se_ref, bF_ref, W1_ref, b1_ref, g1_ref,
               be1_ref, W2_ref, b2_ref, g2_ref, be2_ref, W3T_ref, REP_ref,
               REPM_ref, RED_ref, PK_ref, TILEB_ref, out_ref):
    h = jnp.dot(inv_ref[...], W1_ref[...],
                preferred_element_type=jnp.float32) + b1_ref[...]
    h = jnp.maximum(_ln(h, g1_ref[...], be1_ref[...]), 0.0)
    h = jnp.dot(h, W2_ref[...],
                preferred_element_type=jnp.float32) + b2_ref[...]
    h = jnp.maximum(_ln(h, g2_ref[...], be2_ref[...]), 0.0)
    hrep = jnp.dot(h, REPM_ref[...], preferred_element_type=jnp.float32)

    feat = feat_ref[...]
    bF = bF_ref[...]
    REP = REP_ref[...]
    PK = PK_ref[...]

    Fl = [jnp.dot(feat, REP[l * 64:(l + 1) * 64, :],
                  preferred_element_type=jnp.float32) for l in range(_K)]
    BT = jnp.dot(bF, TILEB_ref[...], preferred_element_type=jnp.float32)

    acc = se_ref[...]
    for k in range(_K):
        T = None
        for l in range(_K):
            j = (l * 4 + k) * 96
            t = Fl[l] * BT[:, j:j + 96]
            T = t if T is None else T + t
        S = jnp.dot(T, W3T_ref[...],
                    preferred_element_type=jnp.float32)
        ok = jnp.dot(S * hrep, RED_ref[...],
                     preferred_element_type=jnp.float32)
        acc = acc + jnp.dot(ok, PK[k * 16:(k + 1) * 16, :],
                            preferred_element_type=jnp.float32)
    out_ref[...] = acc


def _edge_dense(inv, feat_e, se_e, bF, W1, b1, g1, be1, W2, b2, g2, be2, W3T,
                REP, REPM, RED, PK, TILEB):
    nb = _E // _BE
    edge = lambda w: pl.BlockSpec((_BE, w), lambda g: (g, 0))
    full = lambda a, b: pl.BlockSpec((a, b), lambda g: (0, 0))
    return pl.pallas_call(
        _edge_body,
        grid=(nb,),
        in_specs=[
            edge(5), edge(64), edge(64), edge(96),
            full(5, 32), full(1, 32), full(1, 32), full(1, 32),
            full(32, 32), full(1, 32), full(1, 32), full(1, 32),
            full(96, 512), full(256, 96), full(32, 512), full(512, 16),
            full(64, 64), full(96, 1536),
        ],
        out_specs=edge(64),
        out_shape=jax.ShapeDtypeStruct((_E, 64), jnp.float32),
    )(inv, feat_e, se_e, bF, W1, b1, g1, be1, W2, b2, g2, be2, W3T, REP,
      REPM, RED, PK, TILEB)


def _sc_scatter(out_e, dst, zeros_hbm):
    mesh = plsc.VectorSubcoreMesh(core_axis_name="c", subcore_axis_name="s")
    nc, ns = mesh.num_cores, mesh.num_subcores
    nw = nc * ns
    per_w = _E // nw
    n_ch = per_w // _CH
    rows_per_tile = _N // ns

    @functools.partial(
        pl.kernel,
        mesh=mesh,
        compiler_params=pltpu.CompilerParams(use_tc_tiling_on_sc=False),
        out_type=jax.ShapeDtypeStruct((nc, _N, 64), jnp.float32),
        scratch_types=[
            pltpu.VMEM((_CH,), jnp.int32),
            pltpu.VMEM((_CH, 64), jnp.float32),
            pltpu.VMEM_SHARED((_N, 64), jnp.float32),
        ],
    )
    def k(oe_h, dst_h, z_h, out_h, di_v, rows_v, acc_sh):
        cid = lax.axis_index("c")
        sid = lax.axis_index("s")
        wid = sid * nc + cid
        rbase = sid * rows_per_tile
        pltpu.sync_copy(z_h.at[pl.ds(rbase, rows_per_tile)],
                        acc_sh.at[pl.ds(rbase, rows_per_tile)])
        plsc.subcore_barrier()

        def body(i, carry):
            off = wid * per_w + i * _CH
            pltpu.sync_copy(dst_h.at[pl.ds(off, _CH)], di_v)
            pltpu.sync_copy(oe_h.at[pl.ds(off, _CH)], rows_v)
            pltpu.sync_copy(rows_v, acc_sh.at[di_v], add=True)
            return carry

        lax.fori_loop(0, n_ch, body, 0)
        plsc.subcore_barrier()
        pltpu.sync_copy(acc_sh.at[pl.ds(rbase, rows_per_tile)],
                        out_h.at[cid].at[pl.ds(rbase, rows_per_tile)])

    return k(out_e, dst, zeros_hbm)


def _sum_partials_body(p_ref, out_ref):
    out_ref[...] = p_ref[0] + p_ref[1]


def _sum_partials(p):
    nb = _N // _BE
    return pl.pallas_call(
        _sum_partials_body,
        grid=(nb,),
        in_specs=[pl.BlockSpec((2, _BE, 64), lambda g: (0, g, 0))],
        out_specs=pl.BlockSpec((_BE, 64), lambda g: (g, 0)),
        out_shape=jax.ShapeDtypeStruct((_N, 64), jnp.float32),
    )(p)


def _constants():
    REP = np.zeros((_K * 64, _C * _F), np.float32)
    for l in range(_K):
        for c in range(_C):
            for f in range(_F):
                REP[l * 64 + c * 4 + l, c * _F + f] = 1.0
    REPM = np.zeros((_MID, _MID * _C), np.float32)
    for m in range(_MID):
        REPM[m, m * 16:(m + 1) * 16] = 1.0
    RED = np.zeros((_MID * _C, _C), np.float32)
    for m in range(_MID):
        for o in range(_C):
            RED[m * 16 + o, o] = 1.0
    PK = np.zeros((64, 64), np.float32)
    for k in range(_K):
        for o in range(_C):
            PK[k * 16 + o, o * 4 + k] = 1.0
    TILEB = np.zeros((96, 16 * 96), np.float32)
    for l in range(_K):
        for k in range(_K):
            for f in range(_F):
                for c in range(_C):
                    TILEB[l * 24 + f * 4 + k, (l * 4 + k) * 96 + c * 6 + f] = 1.0
    return (jnp.asarray(REP), jnp.asarray(REPM), jnp.asarray(RED),
            jnp.asarray(PK), jnp.asarray(TILEB))


def kernel(node_feats_0, node_feats_1, edge_feats_0, edge_index, basis, W1,
           b1, ln1_g, ln1_b, W2, b2, ln2_g, ln2_b, W3, k_self_0, k_self_1):
    src = edge_index[0]
    dst = edge_index[1]
    inv = edge_feats_0[:, :, 0]

    nfT = jnp.concatenate([node_feats_0, node_feats_1], axis=-1).reshape(_N, 64)

    bF = basis.reshape(_E, 96)

    KS = jnp.zeros((64, 64), jnp.float32)
    for k in range(_K):
        ks = k_self_0 if k == 0 else k_self_1
        KS = KS.at[k::4, k::4].set(ks.T)

    REP, REPM, RED, PK, TILEB = _constants()
    W3T = jnp.transpose(W3.reshape(_MID, _C, 96), (2, 0, 1)).reshape(96, 512)

    s = _self_table(nfT, KS)
    feat_e, se_e = _sc_gather(nfT, s, src, dst)
    out_e = _edge_dense(inv, feat_e, se_e, bF, W1, b1.reshape(1, 32),
                        ln1_g.reshape(1, 32), ln1_b.reshape(1, 32), W2,
                        b2.reshape(1, 32), ln2_g.reshape(1, 32),
                        ln2_b.reshape(1, 32), W3T, REP, REPM, RED, PK, TILEB)
    p = _sc_scatter(out_e, dst, jnp.zeros((_N, 64), jnp.float32))
    res = _sum_partials(p)
    return res.reshape(_N, _C, _K)


# speedup vs baseline: 13.8099x; 1.0182x over previous
import functools

import jax
import jax.numpy as jnp
import numpy as np
from jax import lax
from jax.experimental import pallas as pl
from jax.experimental.pallas import tpu as pltpu
from jax.experimental.pallas import tpu_sc as plsc

_N = 8192
_E = 65536
_C = 16
_MID = 32
_F = 6
_K = 4
_BE = 1024
_CH = 128


def _ln(x, g, b):
    mu = jnp.mean(x, axis=-1, keepdims=True)
    var = jnp.mean((x - mu) ** 2, axis=-1, keepdims=True)
    return (x - mu) / jnp.sqrt(var + 1e-5) * g + b


def _self_table_body(nfT_ref, KS_ref, s_ref):
    s_ref[...] = jnp.dot(nfT_ref[...], KS_ref[...],
                         preferred_element_type=jnp.float32)


def _self_table(nfT, KS):
    nb = _N // _BE
    return pl.pallas_call(
        _self_table_body,
        grid=(nb,),
        in_specs=[
            pl.BlockSpec((_BE, 64), lambda g: (g, 0)),
            pl.BlockSpec((64, 64), lambda g: (0, 0)),
        ],
        out_specs=pl.BlockSpec((_BE, 64), lambda g: (g, 0)),
        out_shape=jax.ShapeDtypeStruct((_N, 64), jnp.float32),
    )(nfT, KS)


def _sc_gather(nfT, s, src, dst):
    mesh = plsc.VectorSubcoreMesh(core_axis_name="c", subcore_axis_name="s")
    nc, ns = mesh.num_cores, mesh.num_subcores
    nw = nc * ns
    per_w = _E // nw
    n_ch = per_w // _CH

    @functools.partial(
        pl.kernel,
        mesh=mesh,
        compiler_params=pltpu.CompilerParams(use_tc_tiling_on_sc=False),
        out_type=(
            jax.ShapeDtypeStruct((_E, 64), jnp.float32),
            jax.ShapeDtypeStruct((_E, 64), jnp.float32),
        ),
        scratch_types=[
            pltpu.VMEM((_CH,), jnp.int32),
            pltpu.VMEM((_CH,), jnp.int32),
            pltpu.VMEM((_CH, 64), jnp.float32),
            pltpu.VMEM((_CH, 64), jnp.float32),
            pltpu.SemaphoreType.DMA,
        ],
    )
    def k(nfT_h, s_h, src_h, dst_h, feat_h, se_h, si_v, di_v, fr_v, sr_v, sem):
        wid = lax.axis_index("s") * nc + lax.axis_index("c")
        base = wid * per_w

        def body(i, carry):
            off = base + i * _CH
            ca = pltpu.async_copy(src_h.at[pl.ds(off, _CH)], si_v, sem)
            cb = pltpu.async_copy(dst_h.at[pl.ds(off, _CH)], di_v, sem)
            ca.wait()
            cb.wait()
            ga = pltpu.async_copy(nfT_h.at[si_v], fr_v, sem)
            gb = pltpu.async_copy(s_h.at[di_v], sr_v, sem)
            ga.wait()
            gb.wait()
            wa = pltpu.async_copy(fr_v, feat_h.at[pl.ds(off, _CH)], sem)
            wb = pltpu.async_copy(sr_v, se_h.at[pl.ds(off, _CH)], sem)
            wa.wait()
            wb.wait()
            return carry

        lax.fori_loop(0, n_ch, body, 0)

    return k(nfT, s, src, dst)


def _edge_body(inv_ref, feat_ref, se_ref, bF_ref, W1_ref, b1_ref, g1_ref,
               be1_ref, W2_ref, b2_ref, g2_ref, be2_ref, W3T_ref, REP_ref,
               REPM_ref, RED_ref, PK_ref, TILEB_ref, out_ref):
    h = jnp.dot(inv_ref[...], W1_ref[...],
                preferred_element_type=jnp.float32) + b1_ref[...]
    h = jnp.maximum(_ln(h, g1_ref[...], be1_ref[...]), 0.0)
    h = jnp.dot(h, W2_ref[...],
                preferred_element_type=jnp.float32) + b2_ref[...]
    h = jnp.maximum(_ln(h, g2_ref[...], be2_ref[...]), 0.0)
    hrep = jnp.dot(h, REPM_ref[...], preferred_element_type=jnp.float32)

    feat = feat_ref[...]
    bF = bF_ref[...]
    REP = REP_ref[...]
    PK = PK_ref[...]

    Fl = [jnp.dot(feat, REP[l * 64:(l + 1) * 64, :],
                  preferred_element_type=jnp.float32) for l in range(_K)]
    BT = jnp.dot(bF, TILEB_ref[...], preferred_element_type=jnp.float32)

    acc = se_ref[...]
    for k in range(_K):
        T = None
        for l in range(_K):
            j = (l * 4 + k) * 96
            t = Fl[l] * BT[:, j:j + 96]
            T = t if T is None else T + t
        S = jnp.dot(T, W3T_ref[...],
                    preferred_element_type=jnp.float32)
        ok = jnp.dot(S * hrep, RED_ref[...],
                     preferred_element_type=jnp.float32)
        acc = acc + jnp.dot(ok, PK[k * 16:(k + 1) * 16, :],
                            preferred_element_type=jnp.float32)
    out_ref[...] = acc


def _edge_dense(inv, feat_e, se_e, bF, W1, b1, g1, be1, W2, b2, g2, be2, W3T,
                REP, REPM, RED, PK, TILEB):
    nb = _E // _BE
    edge = lambda w: pl.BlockSpec((_BE, w), lambda g: (g, 0))
    full = lambda a, b: pl.BlockSpec((a, b), lambda g: (0, 0))
    return pl.pallas_call(
        _edge_body,
        grid=(nb,),
        in_specs=[
            edge(5), edge(64), edge(64), edge(96),
            full(5, 32), full(1, 32), full(1, 32), full(1, 32),
            full(32, 32), full(1, 32), full(1, 32), full(1, 32),
            full(96, 512), full(256, 96), full(32, 512), full(512, 16),
            full(64, 64), full(96, 1536),
        ],
        out_specs=edge(64),
        out_shape=jax.ShapeDtypeStruct((_E, 64), jnp.float32),
    )(inv, feat_e, se_e, bF, W1, b1, g1, be1, W2, b2, g2, be2, W3T, REP,
      REPM, RED, PK, TILEB)


def _sc_scatter(out_e, dst, zeros_hbm):
    mesh = plsc.VectorSubcoreMesh(core_axis_name="c", subcore_axis_name="s")
    nc, ns = mesh.num_cores, mesh.num_subcores
    nw = nc * ns
    per_w = _E // nw
    n_ch = per_w // _CH
    rows_per_tile = _N // ns

    @functools.partial(
        pl.kernel,
        mesh=mesh,
        compiler_params=pltpu.CompilerParams(use_tc_tiling_on_sc=False),
        out_type=jax.ShapeDtypeStruct((nc, _N, 64), jnp.float32),
        scratch_types=[
            pltpu.VMEM((_CH,), jnp.int32),
            pltpu.VMEM((_CH, 64), jnp.float32),
            pltpu.VMEM_SHARED((_N, 64), jnp.float32),
            pltpu.SemaphoreType.DMA,
        ],
    )
    def k(oe_h, dst_h, z_h, out_h, di_v, rows_v, acc_sh, sem):
        cid = lax.axis_index("c")
        sid = lax.axis_index("s")
        wid = sid * nc + cid
        rbase = sid * rows_per_tile
        pltpu.sync_copy(z_h.at[pl.ds(rbase, rows_per_tile)],
                        acc_sh.at[pl.ds(rbase, rows_per_tile)])
        plsc.subcore_barrier()

        def body(i, carry):
            off = wid * per_w + i * _CH
            ca = pltpu.async_copy(dst_h.at[pl.ds(off, _CH)], di_v, sem)
            cb = pltpu.async_copy(oe_h.at[pl.ds(off, _CH)], rows_v, sem)
            ca.wait()
            cb.wait()
            pltpu.sync_copy(rows_v, acc_sh.at[di_v], add=True)
            return carry

        lax.fori_loop(0, n_ch, body, 0)
        plsc.subcore_barrier()
        pltpu.sync_copy(acc_sh.at[pl.ds(rbase, rows_per_tile)],
                        out_h.at[cid].at[pl.ds(rbase, rows_per_tile)])

    return k(out_e, dst, zeros_hbm)


def _sum_partials_body(p_ref, out_ref):
    out_ref[...] = p_ref[0] + p_ref[1]


def _sum_partials(p):
    nb = _N // _BE
    return pl.pallas_call(
        _sum_partials_body,
        grid=(nb,),
        in_specs=[pl.BlockSpec((2, _BE, 64), lambda g: (0, g, 0))],
        out_specs=pl.BlockSpec((_BE, 64), lambda g: (g, 0)),
        out_shape=jax.ShapeDtypeStruct((_N, 64), jnp.float32),
    )(p)


def _constants():
    REP = np.zeros((_K * 64, _C * _F), np.float32)
    for l in range(_K):
        for c in range(_C):
            for f in range(_F):
                REP[l * 64 + c * 4 + l, c * _F + f] = 1.0
    REPM = np.zeros((_MID, _MID * _C), np.float32)
    for m in range(_MID):
        REPM[m, m * 16:(m + 1) * 16] = 1.0
    RED = np.zeros((_MID * _C, _C), np.float32)
    for m in range(_MID):
        for o in range(_C):
            RED[m * 16 + o, o] = 1.0
    PK = np.zeros((64, 64), np.float32)
    for k in range(_K):
        for o in range(_C):
            PK[k * 16 + o, o * 4 + k] = 1.0
    TILEB = np.zeros((96, 16 * 96), np.float32)
    for l in range(_K):
        for k in range(_K):
            for f in range(_F):
                for c in range(_C):
                    TILEB[l * 24 + f * 4 + k, (l * 4 + k) * 96 + c * 6 + f] = 1.0
    return (jnp.asarray(REP), jnp.asarray(REPM), jnp.asarray(RED),
            jnp.asarray(PK), jnp.asarray(TILEB))


def kernel(node_feats_0, node_feats_1, edge_feats_0, edge_index, basis, W1,
           b1, ln1_g, ln1_b, W2, b2, ln2_g, ln2_b, W3, k_self_0, k_self_1):
    src = edge_index[0]
    dst = edge_index[1]
    inv = edge_feats_0[:, :, 0]

    nfT = jnp.concatenate([node_feats_0, node_feats_1], axis=-1).reshape(_N, 64)

    bF = basis.reshape(_E, 96)

    KS = jnp.zeros((64, 64), jnp.float32)
    for k in range(_K):
        ks = k_self_0 if k == 0 else k_self_1
        KS = KS.at[k::4, k::4].set(ks.T)

    REP, REPM, RED, PK, TILEB = _constants()
    W3T = jnp.transpose(W3.reshape(_MID, _C, 96), (2, 0, 1)).reshape(96, 512)

    s = _self_table(nfT, KS)
    feat_e, se_e = _sc_gather(nfT, s, src, dst)
    out_e = _edge_dense(inv, feat_e, se_e, bF, W1, b1.reshape(1, 32),
                        ln1_g.reshape(1, 32), ln1_b.reshape(1, 32), W2,
                        b2.reshape(1, 32), ln2_g.reshape(1, 32),
                        ln2_b.reshape(1, 32), W3T, REP, REPM, RED, PK, TILEB)
    p = _sc_scatter(out_e, dst, jnp.zeros((_N, 64), jnp.float32))
    res = _sum_partials(p)
    return res.reshape(_N, _C, _K)


# speedup vs baseline: 14.5759x; 1.0555x over previous
import functools

import jax
import jax.numpy as jnp
import numpy as np
from jax import lax
from jax.experimental import pallas as pl
from jax.experimental.pallas import tpu as pltpu
from jax.experimental.pallas import tpu_sc as plsc

_N = 8192
_E = 65536
_C = 16
_MID = 32
_F = 6
_K = 4
_BE = 2048
_CH = 128


def _ln(x, g, b):
    mu = jnp.mean(x, axis=-1, keepdims=True)
    var = jnp.mean((x - mu) ** 2, axis=-1, keepdims=True)
    return (x - mu) / jnp.sqrt(var + 1e-5) * g + b


def _self_table_body(nfT_ref, KS_ref, s_ref):
    s_ref[...] = jnp.dot(nfT_ref[...], KS_ref[...],
                         preferred_element_type=jnp.float32)


def _self_table(nfT, KS):
    nb = _N // _BE
    return pl.pallas_call(
        _self_table_body,
        grid=(nb,),
        in_specs=[
            pl.BlockSpec((_BE, 64), lambda g: (g, 0)),
            pl.BlockSpec((64, 64), lambda g: (0, 0)),
        ],
        out_specs=pl.BlockSpec((_BE, 64), lambda g: (g, 0)),
        out_shape=jax.ShapeDtypeStruct((_N, 64), jnp.float32),
    )(nfT, KS)


def _sc_gather(nfT, s, src, dst):
    mesh = plsc.VectorSubcoreMesh(core_axis_name="c", subcore_axis_name="s")
    nc, ns = mesh.num_cores, mesh.num_subcores
    nw = nc * ns
    per_w = _E // nw
    n_ch = per_w // _CH

    @functools.partial(
        pl.kernel,
        mesh=mesh,
        compiler_params=pltpu.CompilerParams(use_tc_tiling_on_sc=False),
        out_type=(
            jax.ShapeDtypeStruct((_E, 64), jnp.float32),
            jax.ShapeDtypeStruct((_E, 64), jnp.float32),
        ),
        scratch_types=[
            pltpu.VMEM((_CH,), jnp.int32),
            pltpu.VMEM((_CH,), jnp.int32),
            pltpu.VMEM((_CH, 64), jnp.float32),
            pltpu.VMEM((_CH, 64), jnp.float32),
            pltpu.SemaphoreType.DMA,
        ],
    )
    def k(nfT_h, s_h, src_h, dst_h, feat_h, se_h, si_v, di_v, fr_v, sr_v, sem):
        wid = lax.axis_index("s") * nc + lax.axis_index("c")
        base = wid * per_w

        def body(i, carry):
            off = base + i * _CH
            ca = pltpu.async_copy(src_h.at[pl.ds(off, _CH)], si_v, sem)
            cb = pltpu.async_copy(dst_h.at[pl.ds(off, _CH)], di_v, sem)
            ca.wait()
            cb.wait()
            ga = pltpu.async_copy(nfT_h.at[si_v], fr_v, sem)
            gb = pltpu.async_copy(s_h.at[di_v], sr_v, sem)
            ga.wait()
            gb.wait()
            wa = pltpu.async_copy(fr_v, feat_h.at[pl.ds(off, _CH)], sem)
            wb = pltpu.async_copy(sr_v, se_h.at[pl.ds(off, _CH)], sem)
            wa.wait()
            wb.wait()
            return carry

        lax.fori_loop(0, n_ch, body, 0)

    return k(nfT, s, src, dst)


def _edge_body(inv_ref, feat_ref, se_ref, bF_ref, W1_ref, b1_ref, g1_ref,
               be1_ref, W2_ref, b2_ref, g2_ref, be2_ref, W3T_ref, REP_ref,
               REPM_ref, RED_ref, PK_ref, TILEB_ref, out_ref):
    h = jnp.dot(inv_ref[...], W1_ref[...],
                preferred_element_type=jnp.float32) + b1_ref[...]
    h = jnp.maximum(_ln(h, g1_ref[...], be1_ref[...]), 0.0)
    h = jnp.dot(h, W2_ref[...],
                preferred_element_type=jnp.float32) + b2_ref[...]
    h = jnp.maximum(_ln(h, g2_ref[...], be2_ref[...]), 0.0)
    hrep = jnp.dot(h, REPM_ref[...], preferred_element_type=jnp.float32)

    feat = feat_ref[...]
    bF = bF_ref[...]
    REP = REP_ref[...]
    PK = PK_ref[...]

    Fl = [jnp.dot(feat, REP[l * 64:(l + 1) * 64, :],
                  preferred_element_type=jnp.float32) for l in range(_K)]
    BT = jnp.dot(bF, TILEB_ref[...], preferred_element_type=jnp.float32)

    acc = se_ref[...]
    for k in range(_K):
        T = None
        for l in range(_K):
            j = (l * 4 + k) * 96
            t = Fl[l] * BT[:, j:j + 96]
            T = t if T is None else T + t
        S = jnp.dot(T, W3T_ref[...],
                    preferred_element_type=jnp.float32)
        ok = jnp.dot(S * hrep, RED_ref[...],
                     preferred_element_type=jnp.float32)
        acc = acc + jnp.dot(ok, PK[k * 16:(k + 1) * 16, :],
                            preferred_element_type=jnp.float32)
    out_ref[...] = acc


def _edge_dense(inv, feat_e, se_e, bF, W1, b1, g1, be1, W2, b2, g2, be2, W3T,
                REP, REPM, RED, PK, TILEB):
    nb = _E // _BE
    edge = lambda w: pl.BlockSpec((_BE, w), lambda g: (g, 0))
    full = lambda a, b: pl.BlockSpec((a, b), lambda g: (0, 0))
    return pl.pallas_call(
        _edge_body,
        grid=(nb,),
        in_specs=[
            edge(5), edge(64), edge(64), edge(96),
            full(5, 32), full(1, 32), full(1, 32), full(1, 32),
            full(32, 32), full(1, 32), full(1, 32), full(1, 32),
            full(96, 512), full(256, 96), full(32, 512), full(512, 16),
            full(64, 64), full(96, 1536),
        ],
        out_specs=edge(64),
        out_shape=jax.ShapeDtypeStruct((_E, 64), jnp.float32),
    )(inv, feat_e, se_e, bF, W1, b1, g1, be1, W2, b2, g2, be2, W3T, REP,
      REPM, RED, PK, TILEB)


def _sc_scatter(out_e, dst, zeros_hbm):
    mesh = plsc.VectorSubcoreMesh(core_axis_name="c", subcore_axis_name="s")
    nc, ns = mesh.num_cores, mesh.num_subcores
    nw = nc * ns
    per_w = _E // nw
    n_ch = per_w // _CH
    rows_per_tile = _N // ns

    @functools.partial(
        pl.kernel,
        mesh=mesh,
        compiler_params=pltpu.CompilerParams(use_tc_tiling_on_sc=False),
        out_type=jax.ShapeDtypeStruct((nc, _N, 64), jnp.float32),
        scratch_types=[
            pltpu.VMEM((_CH,), jnp.int32),
            pltpu.VMEM((_CH, 64), jnp.float32),
            pltpu.VMEM_SHARED((_N, 64), jnp.float32),
            pltpu.SemaphoreType.DMA,
        ],
    )
    def k(oe_h, dst_h, z_h, out_h, di_v, rows_v, acc_sh, sem):
        cid = lax.axis_index("c")
        sid = lax.axis_index("s")
        wid = sid * nc + cid
        rbase = sid * rows_per_tile
        pltpu.sync_copy(z_h.at[pl.ds(rbase, rows_per_tile)],
                        acc_sh.at[pl.ds(rbase, rows_per_tile)])
        plsc.subcore_barrier()

        def body(i, carry):
            off = wid * per_w + i * _CH
            ca = pltpu.async_copy(dst_h.at[pl.ds(off, _CH)], di_v, sem)
            cb = pltpu.async_copy(oe_h.at[pl.ds(off, _CH)], rows_v, sem)
            ca.wait()
            cb.wait()
            pltpu.sync_copy(rows_v, acc_sh.at[di_v], add=True)
            return carry

        lax.fori_loop(0, n_ch, body, 0)
        plsc.subcore_barrier()
        pltpu.sync_copy(acc_sh.at[pl.ds(rbase, rows_per_tile)],
                        out_h.at[cid].at[pl.ds(rbase, rows_per_tile)])

    return k(out_e, dst, zeros_hbm)


def _sum_partials_body(p_ref, out_ref):
    out_ref[...] = p_ref[0] + p_ref[1]


def _sum_partials(p):
    nb = _N // _BE
    return pl.pallas_call(
        _sum_partials_body,
        grid=(nb,),
        in_specs=[pl.BlockSpec((2, _BE, 64), lambda g: (0, g, 0))],
        out_specs=pl.BlockSpec((_BE, 64), lambda g: (g, 0)),
        out_shape=jax.ShapeDtypeStruct((_N, 64), jnp.float32),
    )(p)


def _constants():
    REP = np.zeros((_K * 64, _C * _F), np.float32)
    for l in range(_K):
        for c in range(_C):
            for f in range(_F):
                REP[l * 64 + c * 4 + l, c * _F + f] = 1.0
    REPM = np.zeros((_MID, _MID * _C), np.float32)
    for m in range(_MID):
        REPM[m, m * 16:(m + 1) * 16] = 1.0
    RED = np.zeros((_MID * _C, _C), np.float32)
    for m in range(_MID):
        for o in range(_C):
            RED[m * 16 + o, o] = 1.0
    PK = np.zeros((64, 64), np.float32)
    for k in range(_K):
        for o in range(_C):
            PK[k * 16 + o, o * 4 + k] = 1.0
    TILEB = np.zeros((96, 16 * 96), np.float32)
    for l in range(_K):
        for k in range(_K):
            for f in range(_F):
                for c in range(_C):
                    TILEB[l * 24 + f * 4 + k, (l * 4 + k) * 96 + c * 6 + f] = 1.0
    return (jnp.asarray(REP), jnp.asarray(REPM), jnp.asarray(RED),
            jnp.asarray(PK), jnp.asarray(TILEB))


def kernel(node_feats_0, node_feats_1, edge_feats_0, edge_index, basis, W1,
           b1, ln1_g, ln1_b, W2, b2, ln2_g, ln2_b, W3, k_self_0, k_self_1):
    src = edge_index[0]
    dst = edge_index[1]
    inv = edge_feats_0[:, :, 0]

    nfT = jnp.concatenate([node_feats_0, node_feats_1], axis=-1).reshape(_N, 64)

    bF = basis.reshape(_E, 96)

    KS = jnp.zeros((64, 64), jnp.float32)
    for k in range(_K):
        ks = k_self_0 if k == 0 else k_self_1
        KS = KS.at[k::4, k::4].set(ks.T)

    REP, REPM, RED, PK, TILEB = _constants()
    W3T = jnp.transpose(W3.reshape(_MID, _C, 96), (2, 0, 1)).reshape(96, 512)

    s = _self_table(nfT, KS)
    feat_e, se_e = _sc_gather(nfT, s, src, dst)
    out_e = _edge_dense(inv, feat_e, se_e, bF, W1, b1.reshape(1, 32),
                        ln1_g.reshape(1, 32), ln1_b.reshape(1, 32), W2,
                        b2.reshape(1, 32), ln2_g.reshape(1, 32),
                        ln2_b.reshape(1, 32), W3T, REP, REPM, RED, PK, TILEB)
    p = _sc_scatter(out_e, dst, jnp.zeros((_N, 64), jnp.float32))
    res = _sum_partials(p)
    return res.reshape(_N, _C, _K)
